# baseline (speedup 1.0000x reference)
import functools

import jax
import jax.numpy as jnp
from jax import lax
from jax.experimental import pallas as pl
from jax.experimental.pallas import tpu as pltpu
from jax.experimental.pallas import tpu_sc as plsc

_RB = 12544


def _pack_geometry(V):
    nblk = -(-V // (4 * _RB))
    V4 = nblk * _RB
    return nblk, V4


@functools.lru_cache(maxsize=None)
def _make_transpose_pack(V, D):
    nblk, V4 = _pack_geometry(V)

    def body(x0, x1, x2, x3, o_ref):
        for k, xk in enumerate((x0, x1, x2, x3)):
            o_ref[:, k * D:(k + 1) * D] = xk[...].T

    def in_spec(k):
        return pl.BlockSpec((D, _RB), lambda i, _k=k: (0, _k * nblk + i))

    return pl.pallas_call(
        body,
        grid=(nblk,),
        in_specs=[in_spec(0), in_spec(1), in_spec(2), in_spec(3)],
        out_specs=pl.BlockSpec((_RB, 4 * D), lambda i: (i, 0)),
        out_shape=jax.ShapeDtypeStruct((V4, 4 * D), jnp.float32),
    )


@functools.lru_cache(maxsize=None)
def _make_gather(B, W):
    info = plsc.get_sparse_core_info()
    NC, NS = info.num_cores, info.num_subcores
    NW = NC * NS
    b_per_w = B // NW
    mesh = plsc.VectorSubcoreMesh(core_axis_name="c", subcore_axis_name="s")

    @functools.partial(
        pl.kernel,
        mesh=mesh,
        compiler_params=pltpu.CompilerParams(use_tc_tiling_on_sc=True),
        out_type=jax.ShapeDtypeStruct((B, W), jnp.float32),
        scratch_types=[
            pltpu.VMEM((b_per_w,), jnp.int32),
            pltpu.VMEM((b_per_w, W), jnp.float32),
            pltpu.SemaphoreType.DMA,
        ],
    )
    def gather(idx_hbm, table_hbm, out_hbm, idx_v, rows_v, sem):
        wid = lax.axis_index("s") * NC + lax.axis_index("c")
        base = wid * b_per_w
        pltpu.sync_copy(idx_hbm.at[pl.ds(base, b_per_w)], idx_v)

        def fire(g, carry):
            vec = idx_v[pl.ds(g * 16, 16)]
            for l in range(16):
                u = vec[l]
                pltpu.make_async_copy(
                    table_hbm.at[pl.ds(u, 1)],
                    rows_v.at[pl.ds(g * 16 + l, 1)],
                    sem,
                ).start()
            return carry

        lax.fori_loop(0, b_per_w // 16, fire, 0)

        def drain(i, carry):
            pltpu.make_async_copy(
                table_hbm.at[pl.ds(0, 1)], rows_v.at[pl.ds(i, 1)], sem
            ).wait()
            return carry

        lax.fori_loop(0, b_per_w, drain, 0, unroll=8)
        pltpu.sync_copy(rows_v, out_hbm.at[pl.ds(base, b_per_w)])

    return gather


def _mlp_body(x4_ref, slot_ref, w1_ref, b1_ref, w2_ref, b2_ref, o_ref):
    x4 = x4_ref[...]
    slot = slot_ref[...]
    D = w1_ref.shape[0]
    x = jnp.where(slot == 0, x4[:, 0:D], 0.0)
    for k in range(1, 4):
        x = jnp.where(slot == k, x4[:, k * D:(k + 1) * D], x)
    h = jnp.dot(x, w1_ref[...], preferred_element_type=jnp.float32)
    h = jnp.maximum(h + b1_ref[...], 0.0)
    o = jnp.dot(h, w2_ref[...], preferred_element_type=jnp.float32)
    o_ref[...] = o + b2_ref[...]


@functools.lru_cache(maxsize=None)
def _make_mlp(B, D, H, O, blk):
    grid = B // blk
    return pl.pallas_call(
        _mlp_body,
        grid=(grid,),
        in_specs=[
            pl.BlockSpec((blk, 4 * D), lambda i: (i, 0)),
            pl.BlockSpec((blk, 1), lambda i: (i, 0)),
            pl.BlockSpec((D, H), lambda i: (0, 0)),
            pl.BlockSpec((1, H), lambda i: (0, 0)),
            pl.BlockSpec((H, O), lambda i: (0, 0)),
            pl.BlockSpec((1, O), lambda i: (0, 0)),
        ],
        out_specs=pl.BlockSpec((blk, O), lambda i: (i, 0)),
        out_shape=jax.ShapeDtypeStruct((B, O), jnp.float32),
    )


def kernel(user_ids, table, W1, b1, W2, b2):
    B = user_ids.shape[0]
    V, D = table.shape
    H = W1.shape[1]
    O = W2.shape[1]
    _, V4 = _pack_geometry(V)
    idx = user_ids.astype(jnp.int32)
    rows = idx % V4
    slot = (idx // V4).reshape(B, 1)
    packed = _make_transpose_pack(V, D)(table.T, table.T, table.T, table.T)
    x4 = _make_gather(B, 4 * D)(rows, packed)
    mlp = _make_mlp(B, D, H, O, 2048)
    return mlp(x4, slot, W1, b1.reshape(1, H), W2, b2.reshape(1, O))


# speedup vs baseline: 2.1054x; 2.1054x over previous
import functools

import jax
import jax.numpy as jnp
from jax import lax
from jax.experimental import pallas as pl
from jax.experimental.pallas import tpu as pltpu
from jax.experimental.pallas import tpu_sc as plsc

_RB = 12544


def _pack_geometry(V):
    nblk = -(-V // (4 * _RB))
    V4 = nblk * _RB
    return nblk, V4


@functools.lru_cache(maxsize=None)
def _make_transpose_pack(V, D):
    nblk, V4 = _pack_geometry(V)

    def body(x0, x1, x2, x3, eye_ref, o_ref):
        X = jnp.concatenate(
            [x0[...], x1[...], x2[...], x3[...]], axis=0
        )
        o_ref[...] = lax.dot_general(
            X, eye_ref[...], (((0,), (0,)), ((), ())),
            preferred_element_type=jnp.float32,
        )

    def in_spec(k):
        return pl.BlockSpec((D, _RB), lambda i, _k=k: (0, _k * nblk + i))

    return pl.pallas_call(
        body,
        grid=(nblk,),
        in_specs=[
            in_spec(0), in_spec(1), in_spec(2), in_spec(3),
            pl.BlockSpec((4 * D, 4 * D), lambda i: (0, 0)),
        ],
        out_specs=pl.BlockSpec((_RB, 4 * D), lambda i: (i, 0)),
        out_shape=jax.ShapeDtypeStruct((V4, 4 * D), jnp.float32),
    )


@functools.lru_cache(maxsize=None)
def _make_gather(B, W):
    info = plsc.get_sparse_core_info()
    NC, NS = info.num_cores, info.num_subcores
    NW = NC * NS
    b_per_w = B // NW
    mesh = plsc.VectorSubcoreMesh(core_axis_name="c", subcore_axis_name="s")

    @functools.partial(
        pl.kernel,
        mesh=mesh,
        compiler_params=pltpu.CompilerParams(use_tc_tiling_on_sc=True),
        out_type=jax.ShapeDtypeStruct((B, W), jnp.float32),
        scratch_types=[
            pltpu.VMEM((b_per_w,), jnp.int32),
            pltpu.VMEM((b_per_w, W), jnp.float32),
            pltpu.SemaphoreType.DMA,
        ],
    )
    def gather(idx_hbm, table_hbm, out_hbm, idx_v, rows_v, sem):
        wid = lax.axis_index("s") * NC + lax.axis_index("c")
        base = wid * b_per_w
        pltpu.sync_copy(idx_hbm.at[pl.ds(base, b_per_w)], idx_v)

        def fire(g, carry):
            vec = idx_v[pl.ds(g * 16, 16)]
            for l in range(16):
                u = vec[l]
                pltpu.make_async_copy(
                    table_hbm.at[pl.ds(u, 1)],
                    rows_v.at[pl.ds(g * 16 + l, 1)],
                    sem,
                ).start()
            return carry

        lax.fori_loop(0, b_per_w // 16, fire, 0)

        def drain(i, carry):
            pltpu.make_async_copy(
                table_hbm.at[pl.ds(0, 1)], rows_v.at[pl.ds(i, 1)], sem
            ).wait()
            return carry

        lax.fori_loop(0, b_per_w, drain, 0, unroll=8)
        pltpu.sync_copy(rows_v, out_hbm.at[pl.ds(base, b_per_w)])

    return gather


def _mlp_body(x4_ref, slot_ref, w1_ref, b1_ref, w2_ref, b2_ref, o_ref):
    x4 = x4_ref[...]
    slot = slot_ref[...]
    D = w1_ref.shape[0]
    x = jnp.where(slot == 0, x4[:, 0:D], 0.0)
    for k in range(1, 4):
        x = jnp.where(slot == k, x4[:, k * D:(k + 1) * D], x)
    h = jnp.dot(x, w1_ref[...], preferred_element_type=jnp.float32)
    h = jnp.maximum(h + b1_ref[...], 0.0)
    o = jnp.dot(h, w2_ref[...], preferred_element_type=jnp.float32)
    o_ref[...] = o + b2_ref[...]


@functools.lru_cache(maxsize=None)
def _make_mlp(B, D, H, O, blk):
    grid = B // blk
    return pl.pallas_call(
        _mlp_body,
        grid=(grid,),
        in_specs=[
            pl.BlockSpec((blk, 4 * D), lambda i: (i, 0)),
            pl.BlockSpec((blk, 1), lambda i: (i, 0)),
            pl.BlockSpec((D, H), lambda i: (0, 0)),
            pl.BlockSpec((1, H), lambda i: (0, 0)),
            pl.BlockSpec((H, O), lambda i: (0, 0)),
            pl.BlockSpec((1, O), lambda i: (0, 0)),
        ],
        out_specs=pl.BlockSpec((blk, O), lambda i: (i, 0)),
        out_shape=jax.ShapeDtypeStruct((B, O), jnp.float32),
    )


def kernel(user_ids, table, W1, b1, W2, b2):
    B = user_ids.shape[0]
    V, D = table.shape
    H = W1.shape[1]
    O = W2.shape[1]
    _, V4 = _pack_geometry(V)
    idx = user_ids.astype(jnp.int32)
    rows = idx % V4
    slot = (idx // V4).reshape(B, 1)
    eye = jnp.eye(4 * D, dtype=jnp.float32)
    packed = _make_transpose_pack(V, D)(table.T, table.T, table.T, table.T, eye)
    x4 = _make_gather(B, 4 * D)(rows, packed)
    mlp = _make_mlp(B, D, H, O, 2048)
    return mlp(x4, slot, W1, b1.reshape(1, H), W2, b2.reshape(1, O))
